# baseline (speedup 1.0000x reference)
import functools

import jax
import jax.numpy as jnp
from jax import lax
from jax.experimental import pallas as pl
from jax.experimental.pallas import tpu as pltpu
from jax.experimental.pallas import tpu_sc as plsc

VOCAB = 100000
WORD_DIM = 252
TEMPL_DIM = 256
OUT_DIM = 512
B, L = 1024, 200
N = B * L
NC, NS = 2, 16
NW = NC * NS
PER_W = N // NW
C = 80
NCHUNK = PER_W // C
NTRIPLE = (NCHUNK - 2) // 3


def _make_sc_kernel():
    mesh = plsc.VectorSubcoreMesh(core_axis_name="c", subcore_axis_name="s")

    @functools.partial(
        pl.kernel,
        mesh=mesh,
        out_type=jax.ShapeDtypeStruct((N, OUT_DIM), jnp.float32),
        scratch_types=[
            pltpu.VMEM((C,), jnp.int32),
            pltpu.VMEM((C,), jnp.int32),
            pltpu.VMEM((C,), jnp.int32),
            pltpu.VMEM((C,), jnp.int32),
            pltpu.VMEM((C,), jnp.int32),
            pltpu.VMEM((C,), jnp.int32),
            pltpu.VMEM((C, OUT_DIM), jnp.float32),
            pltpu.VMEM((C, OUT_DIM), jnp.float32),
            pltpu.VMEM((C, OUT_DIM), jnp.float32),
            pltpu.VMEM((4 * C,), jnp.float32),
            pltpu.VMEM((4 * C,), jnp.float32),
            pltpu.VMEM((4 * C,), jnp.float32),
            pltpu.SemaphoreType.DMA,
            pltpu.SemaphoreType.DMA,
            pltpu.SemaphoreType.DMA,
            pltpu.SemaphoreType.DMA,
            pltpu.SemaphoreType.DMA,
            pltpu.SemaphoreType.DMA,
            pltpu.SemaphoreType.DMA,
            pltpu.SemaphoreType.DMA,
            pltpu.SemaphoreType.DMA,
        ],
        compiler_params=pltpu.CompilerParams(needs_layout_passes=False),
    )
    def emb_concat(cat_hbm, templ_hbm, box_hbm, ww_hbm, wt_hbm, out_hbm,
                   idx_w0, idx_t0, idx_w1, idx_t1, idx_w2, idx_t2,
                   rows0, rows1, rows2, bb0, bb1, bb2,
                   sg0, sg1, sg2, si0, si1, si2, sw0, sw1, sw2):
        wid = lax.axis_index("s") * NC + lax.axis_index("c")
        base0 = wid * PER_W
        sets = ((idx_w0, idx_t0, rows0, bb0, sg0, si0, sw0),
                (idx_w1, idx_t1, rows1, bb1, sg1, si1, sw1),
                (idx_w2, idx_t2, rows2, bb2, sg2, si2, sw2))

        def stage_idx(c, st):
            iw, it, _, _, _, si, _ = st
            base = base0 + c * C
            pltpu.async_copy(cat_hbm.at[pl.ds(base, C)], iw, si)
            pltpu.async_copy(templ_hbm.at[pl.ds(base, C)], it, si)

        def wait_idx(st):
            iw, it, _, _, _, si, _ = st
            pltpu.make_async_copy(cat_hbm.at[pl.ds(0, C)], iw, si).wait()
            pltpu.make_async_copy(templ_hbm.at[pl.ds(0, C)], it, si).wait()

        def start_gathers(c, st):
            iw, it, rows, bb, sg, _, _ = st
            base = base0 + c * C
            pltpu.async_copy(
                ww_hbm.at[iw], rows.at[:, pl.ds(0, TEMPL_DIM)], sg)
            pltpu.async_copy(
                wt_hbm.at[it], rows.at[:, pl.ds(TEMPL_DIM, TEMPL_DIM)], sg)
            pltpu.async_copy(box_hbm.at[pl.ds(base * 4, 4 * C)], bb, sg)

        def wait_gathers(st):
            _, _, rows, bb, sg, _, _ = st
            pltpu.make_async_copy(
                out_hbm.at[pl.ds(0, C), pl.ds(0, TEMPL_DIM)],
                rows.at[:, pl.ds(0, TEMPL_DIM)], sg).wait()
            pltpu.make_async_copy(
                out_hbm.at[pl.ds(0, C), pl.ds(0, TEMPL_DIM)],
                rows.at[:, pl.ds(TEMPL_DIM, TEMPL_DIM)], sg).wait()
            pltpu.make_async_copy(
                box_hbm.at[pl.ds(0, 4 * C)], bb, sg).wait()

        def fixup(st):
            _, _, rows, bb, _, _, _ = st
            lane = lax.iota(jnp.int32, 16)
            row4 = lax.shift_right_logical(lane, 2)
            col4 = lax.bitwise_and(lane, 3)
            for g in range(C // 4):
                rg = row4 + (4 * g)
                th = plsc.load_gather(rows, [rg, col4 + (OUT_DIM - 4)])
                plsc.store_scatter(rows, [rg, col4 + WORD_DIM], th)
                bx = bb[pl.ds(16 * g, 16)]
                plsc.store_scatter(rows, [rg, col4 + (OUT_DIM - 4)], bx)

        def start_write(c, st):
            _, _, rows, _, _, _, sw = st
            base = base0 + c * C
            pltpu.async_copy(rows, out_hbm.at[pl.ds(base, C)], sw)

        def wait_write(st):
            _, _, rows, _, _, _, sw = st
            pltpu.make_async_copy(
                rows, out_hbm.at[pl.ds(0, C)], sw).wait()

        def step(c, st, st2, do_next, guard_write):
            if do_next:
                stage_idx(c + 2, st2)
            wait_gathers(st)
            fixup(st)
            start_write(c, st)
            if do_next:
                if guard_write:
                    wait_write(st2)
                wait_idx(st2)
                start_gathers(c + 2, st2)

        stage_idx(0, sets[0])
        wait_idx(sets[0])
        start_gathers(0, sets[0])
        stage_idx(1, sets[1])
        wait_idx(sets[1])
        start_gathers(1, sets[1])

        def body(t, carry):
            c = 3 * t

            @pl.when(t == 0)
            def _():
                step(c, sets[0], sets[2], True, False)

            @pl.when(t > 0)
            def _():
                step(c, sets[0], sets[2], True, True)

            step(c + 1, sets[1], sets[0], True, True)
            step(c + 2, sets[2], sets[1], True, True)
            return carry

        lax.fori_loop(0, NTRIPLE, body, 0)
        step(NCHUNK - 2, sets[0], sets[2], False, False)
        step(NCHUNK - 1, sets[1], sets[0], False, False)
        wait_write(sets[2])
        wait_write(sets[0])
        wait_write(sets[1])

    return emb_concat


_emb_concat = _make_sc_kernel()


def kernel(cat_ids, box, template, W_word, W_templ):
    cat_flat = cat_ids.reshape(N).astype(jnp.int32)
    templ_flat = template.reshape(N).astype(jnp.int32)
    box_flat = box.reshape(N * 4)
    ww_pad = jnp.pad(W_word, ((0, 0), (0, TEMPL_DIM - WORD_DIM)))
    wt_rot = jnp.concatenate([W_templ[:, 4:], W_templ[:, :4]], axis=1)
    out = _emb_concat(cat_flat, templ_flat, box_flat, ww_pad, wt_rot)
    return out.reshape(B, L, OUT_DIM)


# speedup vs baseline: 1.1527x; 1.1527x over previous
import functools

import jax
import jax.numpy as jnp
from jax import lax
from jax.experimental import pallas as pl
from jax.experimental.pallas import tpu as pltpu
from jax.experimental.pallas import tpu_sc as plsc

VOCAB = 100000
WORD_DIM = 252
TEMPL_DIM = 256
OUT_DIM = 512
B, L = 1024, 200
N = B * L
NC, NS = 2, 16
NW = NC * NS
PER_W = N // NW
C = 80
NCHUNK = PER_W // C
NTRIPLE = (NCHUNK - 2) // 3


def _make_sc_kernel():
    mesh = plsc.VectorSubcoreMesh(core_axis_name="c", subcore_axis_name="s")

    @functools.partial(
        pl.kernel,
        mesh=mesh,
        out_type=jax.ShapeDtypeStruct((N, OUT_DIM), jnp.float32),
        scratch_types=[
            pltpu.VMEM((C,), jnp.int32),
            pltpu.VMEM((C,), jnp.int32),
            pltpu.VMEM((C,), jnp.int32),
            pltpu.VMEM((C,), jnp.int32),
            pltpu.VMEM((C,), jnp.int32),
            pltpu.VMEM((C,), jnp.int32),
            pltpu.VMEM((C, OUT_DIM), jnp.float32),
            pltpu.VMEM((C, OUT_DIM), jnp.float32),
            pltpu.VMEM((C, OUT_DIM), jnp.float32),
            pltpu.VMEM((4 * C,), jnp.float32),
            pltpu.VMEM((4 * C,), jnp.float32),
            pltpu.VMEM((4 * C,), jnp.float32),
            pltpu.SemaphoreType.DMA,
            pltpu.SemaphoreType.DMA,
            pltpu.SemaphoreType.DMA,
            pltpu.SemaphoreType.DMA,
            pltpu.SemaphoreType.DMA,
            pltpu.SemaphoreType.DMA,
            pltpu.SemaphoreType.DMA,
            pltpu.SemaphoreType.DMA,
            pltpu.SemaphoreType.DMA,
        ],
        compiler_params=pltpu.CompilerParams(needs_layout_passes=False),
    )
    def emb_concat(cat_hbm, templ_hbm, box_hbm, ww_hbm, wt_hbm, out_hbm,
                   idx_w0, idx_t0, idx_w1, idx_t1, idx_w2, idx_t2,
                   rows0, rows1, rows2, bb0, bb1, bb2,
                   sg0, sg1, sg2, si0, si1, si2, sw0, sw1, sw2):
        wid = lax.axis_index("s") * NC + lax.axis_index("c")
        base0 = wid * PER_W
        sets = ((idx_w0, idx_t0, rows0, bb0, sg0, si0, sw0),
                (idx_w1, idx_t1, rows1, bb1, sg1, si1, sw1),
                (idx_w2, idx_t2, rows2, bb2, sg2, si2, sw2))

        def stage_idx(c, st):
            iw, it, _, _, _, si, _ = st
            base = base0 + c * C
            pltpu.async_copy(cat_hbm.at[pl.ds(base, C)], iw, si)
            pltpu.async_copy(templ_hbm.at[pl.ds(base, C)], it, si)

        def wait_idx(st):
            iw, it, _, _, _, si, _ = st
            pltpu.make_async_copy(cat_hbm.at[pl.ds(0, C)], iw, si).wait()
            pltpu.make_async_copy(templ_hbm.at[pl.ds(0, C)], it, si).wait()

        def start_gathers(c, st):
            iw, it, rows, bb, sg, _, _ = st
            base = base0 + c * C
            pltpu.async_copy(
                ww_hbm.at[iw], rows.at[:, pl.ds(0, TEMPL_DIM)], sg)
            pltpu.async_copy(
                wt_hbm.at[it], rows.at[:, pl.ds(TEMPL_DIM, TEMPL_DIM)], sg)
            pltpu.async_copy(box_hbm.at[pl.ds(base * 4, 4 * C)], bb, sg)

        def wait_gathers(st):
            _, _, rows, bb, sg, _, _ = st
            pltpu.make_async_copy(
                out_hbm.at[pl.ds(0, C), pl.ds(0, TEMPL_DIM)],
                rows.at[:, pl.ds(0, TEMPL_DIM)], sg).wait()
            pltpu.make_async_copy(
                out_hbm.at[pl.ds(0, C), pl.ds(0, TEMPL_DIM)],
                rows.at[:, pl.ds(TEMPL_DIM, TEMPL_DIM)], sg).wait()
            pltpu.make_async_copy(
                box_hbm.at[pl.ds(0, 4 * C)], bb, sg).wait()

        def fixup(st):
            _, _, rows, bb, _, _, _ = st
            lane = lax.iota(jnp.int32, 16)

            def rbody(r, carry):
                rvec = jnp.broadcast_to(r, (16,))
                for j in range(TEMPL_DIM // 16):
                    v = rows[r, pl.ds(TEMPL_DIM + 16 * j, 16)]
                    plsc.store_scatter(
                        rows, [rvec, lane + (WORD_DIM + 16 * j)], v)
                return carry

            lax.fori_loop(0, C, rbody, 0)
            row4 = lax.shift_right_logical(lane, 2)
            col4 = lax.bitwise_and(lane, 3)
            for g in range(C // 4):
                rg = row4 + (4 * g)
                bx = bb[pl.ds(16 * g, 16)]
                plsc.store_scatter(rows, [rg, col4 + (OUT_DIM - 4)], bx)

        def start_write(c, st):
            _, _, rows, _, _, _, sw = st
            base = base0 + c * C
            pltpu.async_copy(rows, out_hbm.at[pl.ds(base, C)], sw)

        def wait_write(st):
            _, _, rows, _, _, _, sw = st
            pltpu.make_async_copy(
                rows, out_hbm.at[pl.ds(0, C)], sw).wait()

        def step(c, st, st2, do_next, guard_write):
            if do_next:
                stage_idx(c + 2, st2)
            wait_gathers(st)
            fixup(st)
            start_write(c, st)
            if do_next:
                if guard_write:
                    wait_write(st2)
                wait_idx(st2)
                start_gathers(c + 2, st2)

        stage_idx(0, sets[0])
        wait_idx(sets[0])
        start_gathers(0, sets[0])
        stage_idx(1, sets[1])
        wait_idx(sets[1])
        start_gathers(1, sets[1])

        def body(t, carry):
            c = 3 * t

            @pl.when(t == 0)
            def _():
                step(c, sets[0], sets[2], True, False)

            @pl.when(t > 0)
            def _():
                step(c, sets[0], sets[2], True, True)

            step(c + 1, sets[1], sets[0], True, True)
            step(c + 2, sets[2], sets[1], True, True)
            return carry

        lax.fori_loop(0, NTRIPLE, body, 0)
        step(NCHUNK - 2, sets[0], sets[2], False, False)
        step(NCHUNK - 1, sets[1], sets[0], False, False)
        wait_write(sets[2])
        wait_write(sets[0])
        wait_write(sets[1])

    return emb_concat


_emb_concat = _make_sc_kernel()


def kernel(cat_ids, box, template, W_word, W_templ):
    cat_flat = cat_ids.reshape(N).astype(jnp.int32)
    templ_flat = template.reshape(N).astype(jnp.int32)
    box_flat = box.reshape(N * 4)
    ww_pad = jnp.pad(W_word, ((0, 0), (0, TEMPL_DIM - WORD_DIM)))
    out = _emb_concat(cat_flat, templ_flat, box_flat, ww_pad, W_templ)
    return out.reshape(B, L, OUT_DIM)
